# Initial kernel scaffold; baseline (speedup 1.0000x reference)
import jax
import jax.numpy as jnp
from jax.experimental import pallas as pl


def kernel(sequences, embedding):
    raise NotImplementedError("write your pallas kernel here")


# speedup vs baseline: 1.0138x; 1.0138x over previous
import functools

import jax
import jax.numpy as jnp
from jax import lax
from jax.experimental import pallas as pl
from jax.experimental.pallas import tpu as pltpu
from jax.experimental.pallas import tpu_sc as plsc

_V = 1000
_D = 1000
_B = 1024 * 50
_NW = 32
_BPW = _B // _NW
_CHUNK = 64
_NCHUNK = _BPW // _CHUNK
_IDX_ROWS_PW = _BPW // _CHUNK


def _gather_kernel(idx_hbm, table_hbm, out_hbm, idx_v, rows_v, gsem):
    c = lax.axis_index("c")
    s = lax.axis_index("s")
    wid = s * 2 + c
    base = wid * _BPW
    pltpu.sync_copy(idx_hbm.at[wid], idx_v)

    def body(j, carry):
        pltpu.async_copy(table_hbm.at[idx_v.at[j]], rows_v, gsem).wait()
        pltpu.sync_copy(rows_v, out_hbm.at[pl.ds(base + j * _CHUNK, _CHUNK)])
        return carry

    lax.fori_loop(0, _NCHUNK, body, 0)


@jax.jit
def _embedding_gather(idx2d, embedding):
    mesh = plsc.VectorSubcoreMesh(core_axis_name="c", subcore_axis_name="s")
    run = functools.partial(
        pl.kernel,
        mesh=mesh,
        out_type=jax.ShapeDtypeStruct((_B, _D), jnp.float32),
        scratch_types=[
            pltpu.VMEM((_IDX_ROWS_PW, _CHUNK), jnp.int32),
            pltpu.VMEM((_CHUNK, _D), jnp.float32),
            pltpu.SemaphoreType.DMA,
        ],
        compiler_params=pltpu.CompilerParams(use_tc_tiling_on_sc=False),
    )(_gather_kernel)
    return run(idx2d, embedding)


def kernel(sequences, embedding):
    idx2d = sequences.reshape(_NW, _IDX_ROWS_PW, _CHUNK).astype(jnp.int32)
    out = _embedding_gather(idx2d, embedding)
    return out.reshape(sequences.shape[0], sequences.shape[1], _D)


# speedup vs baseline: 1.0249x; 1.0110x over previous
import functools

import jax
import jax.numpy as jnp
from jax import lax
from jax.experimental import pallas as pl
from jax.experimental.pallas import tpu as pltpu
from jax.experimental.pallas import tpu_sc as plsc

_V = 1000
_D = 1000
_B = 1024 * 50
_NW = 32
_BPW = _B // _NW
_CHUNK = 32
_NCHUNK = _BPW // _CHUNK
_NBUF = 2


def _gather_kernel(idx_hbm, table_hbm, out_hbm, idx_v, buf0, buf1,
                   g0, g1, s0, s1):
    c = lax.axis_index("c")
    s = lax.axis_index("s")
    wid = s * 2 + c
    base = wid * _BPW
    bufs = (buf0, buf1)
    gsems = (g0, g1)
    ssems = (s0, s1)

    pltpu.sync_copy(idx_hbm.at[wid], idx_v)

    def gather(chunk, b):
        return pltpu.async_copy(table_hbm.at[idx_v.at[chunk]], bufs[b],
                                gsems[b])

    def scatter(chunk, b):
        return pltpu.async_copy(
            bufs[b], out_hbm.at[pl.ds(base + chunk * _CHUNK, _CHUNK)],
            ssems[b])

    for b in range(_NBUF):
        gather(b, b)
    for b in range(_NBUF):
        gather_done = pltpu.make_async_copy(table_hbm.at[idx_v.at[b]],
                                            bufs[b], gsems[b])
        gather_done.wait()
        scatter(b, b)

    def body(g, carry):
        for b in range(_NBUF):
            chunk = g * _NBUF + b
            pltpu.make_async_copy(
                bufs[b], out_hbm.at[pl.ds(base, _CHUNK)], ssems[b]).wait()
            gather(chunk, b)
            pltpu.make_async_copy(table_hbm.at[idx_v.at[chunk]], bufs[b],
                                  gsems[b]).wait()
            scatter(chunk, b)
        return carry

    lax.fori_loop(1, _NCHUNK // _NBUF, body, 0)

    for b in range(_NBUF):
        pltpu.make_async_copy(
            bufs[b], out_hbm.at[pl.ds(base, _CHUNK)], ssems[b]).wait()


@jax.jit
def _embedding_gather(idx3d, embedding):
    mesh = plsc.VectorSubcoreMesh(core_axis_name="c", subcore_axis_name="s")
    run = functools.partial(
        pl.kernel,
        mesh=mesh,
        out_type=jax.ShapeDtypeStruct((_B, _D), jnp.float32),
        scratch_types=[
            pltpu.VMEM((_NCHUNK, _CHUNK), jnp.int32),
            pltpu.VMEM((_CHUNK, _D), jnp.float32),
            pltpu.VMEM((_CHUNK, _D), jnp.float32),
            pltpu.SemaphoreType.DMA,
            pltpu.SemaphoreType.DMA,
            pltpu.SemaphoreType.DMA,
            pltpu.SemaphoreType.DMA,
        ],
        compiler_params=pltpu.CompilerParams(use_tc_tiling_on_sc=False),
    )(_gather_kernel)
    return run(idx3d, embedding)


def kernel(sequences, embedding):
    idx3d = sequences.reshape(_NW, _NCHUNK, _CHUNK).astype(jnp.int32)
    out = _embedding_gather(idx3d, embedding)
    return out.reshape(sequences.shape[0], sequences.shape[1], _D)


# speedup vs baseline: 1.3308x; 1.2984x over previous
import functools

import jax
import jax.numpy as jnp
from jax import lax
from jax.experimental import pallas as pl
from jax.experimental.pallas import tpu as pltpu
from jax.experimental.pallas import tpu_sc as plsc

_V = 1000
_D = 1000
_DP = 1024
_S = 1024
_T = 50
_NW = 32
_SPW = _S // _NW
_G = 16
_NGF = _T // _G
_REM = _T - _NGF * _G
_GPS = _NGF + 1
_L = 16
_NVEC = 63


def _narrow_rows(src, dst, nrows):
    def row(r, carry):
        for m in range(_NVEC - 1):
            dst[r, pl.ds(m * _L, _L)] = src[r, pl.ds(m * _L, _L)]
        dst[r, pl.ds(_D - _L, _L)] = src[r, pl.ds(_D - _L, _L)]
        return carry

    lax.fori_loop(0, nrows, row, 0)


def _gather_kernel(idx_hbm, table_hbm, out_hbm, idx_v, buf0, buf1,
                   nar0, nar1, rb0, rb1, rn0, rn1, g0, g1, s0, s1):
    c = lax.axis_index("c")
    s = lax.axis_index("s")
    wid = s * 2 + c
    i0 = wid * _SPW
    pltpu.sync_copy(idx_hbm.at[pl.ds(wid * _SPW * _GPS, _SPW * _GPS)], idx_v)

    bufs = (buf0, buf1)
    nars = (nar0, nar1)
    rembufs = (rb0, rb1)
    remnars = (rn0, rn1)
    gsems = (g0, g1)
    ssems = (s0, s1)

    def gather(k, b):
        slab = k // _GPS
        jg = k % _GPS

        def full(_):
            pltpu.async_copy(table_hbm.at[idx_v.at[slab * _GPS + jg]],
                             bufs[b], gsems[b])
            return 0

        def remd(_):
            pltpu.async_copy(
                table_hbm.at[idx_v.at[slab * _GPS + _NGF, pl.ds(0, _REM)]],
                rembufs[b], gsems[b])
            return 0

        if isinstance(k, int):
            (full if jg < _NGF else remd)(0)
        else:
            lax.cond(jg < _NGF, full, remd, 0)

    def wait_gather(k, b):
        slab = k // _GPS
        jg = k % _GPS

        def full(_):
            pltpu.make_async_copy(
                table_hbm.at[idx_v.at[slab * _GPS + jg]], bufs[b],
                gsems[b]).wait()
            return 0

        def remd(_):
            pltpu.make_async_copy(
                table_hbm.at[idx_v.at[slab * _GPS + _NGF, pl.ds(0, _REM)]],
                rembufs[b], gsems[b]).wait()
            return 0

        if isinstance(k, int):
            (full if jg < _NGF else remd)(0)
        else:
            lax.cond(jg < _NGF, full, remd, 0)

    def narrow(k, b):
        jg = k % _GPS
        if isinstance(k, int):
            if jg < _NGF:
                _narrow_rows(bufs[b], nars[b], _G)
            else:
                _narrow_rows(rembufs[b], remnars[b], _REM)
        else:
            def full(_):
                _narrow_rows(bufs[b], nars[b], _G)
                return 0

            def remd(_):
                _narrow_rows(rembufs[b], remnars[b], _REM)
                return 0

            lax.cond(jg < _NGF, full, remd, 0)

    def scatter(k, b):
        slab = k // _GPS
        jg = k % _GPS

        def full(_):
            pltpu.async_copy(
                nars[b].at[:, :],
                out_hbm.at[i0 + slab, pl.ds(jg * _G, _G), :], ssems[b])
            return 0

        def remd(_):
            pltpu.async_copy(
                remnars[b],
                out_hbm.at[i0 + slab, pl.ds(_NGF * _G, _REM), :], ssems[b])
            return 0

        if isinstance(k, int):
            (full if jg < _NGF else remd)(0)
        else:
            lax.cond(jg < _NGF, full, remd, 0)

    def wait_scatter(k, b):
        slab = k // _GPS
        jg = k % _GPS

        def full(_):
            pltpu.make_async_copy(
                nars[b].at[:, :],
                out_hbm.at[i0 + slab, pl.ds(jg * _G, _G), :], ssems[b]).wait()
            return 0

        def remd(_):
            pltpu.make_async_copy(
                remnars[b],
                out_hbm.at[i0 + slab, pl.ds(_NGF * _G, _REM), :],
                ssems[b]).wait()
            return 0

        if isinstance(k, int):
            (full if jg < _NGF else remd)(0)
        else:
            lax.cond(jg < _NGF, full, remd, 0)

    n_steps = _SPW * _GPS

    gather(0, 0)
    gather(1, 1)
    wait_gather(0, 0)
    narrow(0, 0)
    scatter(0, 0)
    wait_gather(1, 1)
    narrow(1, 1)
    scatter(1, 1)

    def body(p, carry):
        for b in range(2):
            k = p * 2 + b
            wait_scatter(k - 2, b)
            gather(k, b)
            wait_gather(k, b)
            narrow(k, b)
            scatter(k, b)
        return carry

    lax.fori_loop(1, n_steps // 2, body, 0)

    wait_scatter(n_steps - 2, 0)
    wait_scatter(n_steps - 1, 1)


@jax.jit
def _embedding_gather(idx_groups, table_padded):
    mesh = plsc.VectorSubcoreMesh(core_axis_name="c", subcore_axis_name="s")
    run = functools.partial(
        pl.kernel,
        mesh=mesh,
        out_type=jax.ShapeDtypeStruct((_S, _T, _D), jnp.float32),
        scratch_types=[
            pltpu.VMEM((_SPW * _GPS, _G), jnp.int32),
            pltpu.VMEM((_G, _DP), jnp.float32),
            pltpu.VMEM((_G, _DP), jnp.float32),
            pltpu.VMEM((_G, _D), jnp.float32),
            pltpu.VMEM((_G, _D), jnp.float32),
            pltpu.VMEM((_REM, _DP), jnp.float32),
            pltpu.VMEM((_REM, _DP), jnp.float32),
            pltpu.VMEM((_REM, _D), jnp.float32),
            pltpu.VMEM((_REM, _D), jnp.float32),
            pltpu.SemaphoreType.DMA,
            pltpu.SemaphoreType.DMA,
            pltpu.SemaphoreType.DMA,
            pltpu.SemaphoreType.DMA,
        ],
    )(_gather_kernel)
    return run(idx_groups, table_padded)


def kernel(sequences, embedding):
    seq = sequences.astype(jnp.int32)
    full = seq[:, : _NGF * _G].reshape(_S, _NGF, _G)
    rem = jnp.concatenate(
        [seq[:, _NGF * _G :],
         jnp.tile(seq[:, -1:], (1, _G - _REM))], axis=1).reshape(_S, 1, _G)
    idx_groups = jnp.concatenate([full, rem], axis=1).reshape(_S * _GPS, _G)
    table_padded = jnp.pad(embedding, ((0, 0), (0, _DP - _D)))
    return _embedding_gather(idx_groups, table_padded)
